# baseline (speedup 1.0000x reference)
import functools
import math

import jax
import jax.numpy as jnp
from jax import lax
from jax.experimental import pallas as pl
from jax.experimental.pallas import tpu as pltpu
from jax.experimental.pallas import tpu_sc as plsc


def _make_embed_kernel(n_total, d_model, n_per_w, chunk, num_cores, scale):
    n_chunks = n_per_w // chunk
    mesh = plsc.VectorSubcoreMesh(core_axis_name="c", subcore_axis_name="s")

    @functools.partial(
        pl.kernel,
        mesh=mesh,
        out_type=jax.ShapeDtypeStruct((n_total, d_model), jnp.float32),
        compiler_params=pltpu.CompilerParams(use_tc_tiling_on_sc=False),
        scratch_types=[
            pltpu.VMEM((n_per_w,), jnp.int32),
            pltpu.VMEM((chunk, d_model), jnp.float32),
            pltpu.SemaphoreType.DMA,
            pltpu.SemaphoreType.DMA,
        ],
    )
    def k(idx_hbm, table_hbm, out_hbm, idx_v, rows, gsem, osem):
        wid = lax.axis_index("s") * num_cores + lax.axis_index("c")
        base = wid * n_per_w
        pltpu.sync_copy(idx_hbm.at[pl.ds(base, n_per_w)], idx_v)

        def chunk_body(g, _):
            off = g * chunk
            pltpu.async_copy(
                table_hbm.at[idx_v.at[pl.ds(off, chunk)]], rows, gsem
            ).wait()

            def row_body(j, _):
                for t in range(d_model // 16):
                    sl = pl.ds(t * 16, 16)
                    rows[j, sl] = rows[j, sl] * scale
                return 0

            lax.fori_loop(0, chunk, row_body, 0)
            pltpu.async_copy(
                rows, out_hbm.at[pl.ds(base + off, chunk)], osem
            ).wait()
            return 0

        lax.fori_loop(0, n_chunks, chunk_body, 0)

    return k


def kernel(x, table):
    b, l = x.shape
    v, d = table.shape
    n_total = b * l
    idx = x.reshape(n_total).astype(jnp.int32)
    info = plsc.get_sparse_core_info()
    nw = info.num_cores * info.num_subcores
    n_per_w = n_total // nw
    k = _make_embed_kernel(
        n_total, d, n_per_w, 128, info.num_cores, float(math.sqrt(d))
    )
    out = k(idx, table)
    return out.reshape(b, l, d)
